# Initial kernel scaffold; baseline (speedup 1.0000x reference)
import jax
import jax.numpy as jnp
from jax.experimental import pallas as pl


def kernel(user_ids, item_ids, sample_weight, user_tables, item_tables, uW1, ub1, ug1, ube1, uW2, ub2, ug2, ube2, iW1, ib1, ig1, ibe1, iW2, ib2, ig2, ibe2):
    raise NotImplementedError("write your pallas kernel here")


# speedup vs baseline: 8.2460x; 8.2460x over previous
import functools

import jax
import jax.numpy as jnp
from jax import lax
from jax.experimental import pallas as pl
from jax.experimental.pallas import tpu as pltpu
from jax.experimental.pallas import tpu_sc as plsc

B = 16384
V = 100000
D = 32
NU = 8
NI = 4

NC = 2
NS = 16
NW = NC * NS

CH = 128
GRP = 8
U_ROWS = B * NU
I_ROWS = B * NI
U_PW = U_ROWS // NW
I_PW = I_ROWS // NW
U_CHUNKS = U_PW // CH
I_CHUNKS = I_PW // CH

BM = 2048
NB = B // BM


def _sc_gather_body(utab, uidx, itab, iidx, ue, ie, idx_v, rows_v, sem):
    wid = lax.axis_index("s") * NC + lax.axis_index("c")

    def tower(tab, idx_hbm, out_hbm, n_chunks, rows_pw):
        pltpu.sync_copy(idx_hbm.at[pl.ds(wid * n_chunks, n_chunks)],
                        idx_v.at[pl.ds(0, n_chunks)])

        def group(g, _):
            cps = []
            for s in range(GRP):
                c = g * GRP + s
                cps.append(pltpu.async_copy(tab.at[idx_v.at[c]],
                                            rows_v.at[s], sem))
            for cp in cps:
                cp.wait()
            for s in range(GRP):
                c = g * GRP + s
                pltpu.sync_copy(rows_v.at[s],
                                out_hbm.at[pl.ds(wid * rows_pw + c * CH, CH)])
            return _

        lax.fori_loop(0, n_chunks // GRP, group, 0)

    tower(utab, uidx, ue, U_CHUNKS, U_PW)
    tower(itab, iidx, ie, I_CHUNKS, I_PW)


def _gather_embeddings(utab, uidx, itab, iidx):
    mesh = plsc.VectorSubcoreMesh(core_axis_name="c", subcore_axis_name="s")
    f = functools.partial(
        pl.kernel,
        out_type=[jax.ShapeDtypeStruct((U_ROWS, D), jnp.float32),
                  jax.ShapeDtypeStruct((I_ROWS, D), jnp.float32)],
        mesh=mesh,
        scratch_types=[pltpu.VMEM((U_CHUNKS, CH), jnp.int32),
                       pltpu.VMEM((GRP, CH, D), jnp.float32),
                       pltpu.SemaphoreType.DMA],
        compiler_params=pltpu.CompilerParams(use_tc_tiling_on_sc=False),
    )(_sc_gather_body)
    return f(utab, uidx, itab, iidx)


def _mm_stats_body(x_ref, w_ref, b_ref, z_ref, st_ref):
    z = jnp.dot(x_ref[...], w_ref[...],
                preferred_element_type=jnp.float32) + b_ref[...]
    z_ref[...] = z

    @pl.when(pl.program_id(0) == 0)
    def _():
        st_ref[...] = jnp.zeros_like(st_ref)

    st_ref[...] += jnp.concatenate(
        [jnp.sum(z, axis=0, keepdims=True),
         jnp.sum(z * z, axis=0, keepdims=True)], axis=0)


def _mm_stats(x, w, b):
    Bn, K = x.shape
    N = w.shape[1]
    return pl.pallas_call(
        _mm_stats_body,
        grid=(Bn // BM,),
        in_specs=[pl.BlockSpec((BM, K), lambda i: (i, 0)),
                  pl.BlockSpec((K, N), lambda i: (0, 0)),
                  pl.BlockSpec((1, N), lambda i: (0, 0))],
        out_specs=[pl.BlockSpec((BM, N), lambda i: (i, 0)),
                   pl.BlockSpec((2, N), lambda i: (0, 0))],
        out_shape=[jax.ShapeDtypeStruct((Bn, N), jnp.float32),
                   jax.ShapeDtypeStruct((2, N), jnp.float32)],
    )(x, w, b)


def _bn(z, st, g, be):
    mu = st[0:1, :] * (1.0 / B)
    var = st[1:2, :] * (1.0 / B) - mu * mu
    return (z - mu) / jnp.sqrt(var + 1e-5) * g + be


def _bn_mm_stats_body(z_ref, st_ref, g_ref, be_ref, w_ref, b_ref,
                      z2_ref, st2_ref):
    h = jnp.maximum(_bn(z_ref[...], st_ref[...], g_ref[...], be_ref[...]), 0.0)
    z2 = jnp.dot(h, w_ref[...],
                 preferred_element_type=jnp.float32) + b_ref[...]
    z2_ref[...] = z2

    @pl.when(pl.program_id(0) == 0)
    def _():
        st2_ref[...] = jnp.zeros_like(st2_ref)

    st2_ref[...] += jnp.concatenate(
        [jnp.sum(z2, axis=0, keepdims=True),
         jnp.sum(z2 * z2, axis=0, keepdims=True)], axis=0)


def _bn_mm_stats(z, st, g, be, w, b):
    Bn, K = z.shape
    N = w.shape[1]
    return pl.pallas_call(
        _bn_mm_stats_body,
        grid=(Bn // BM,),
        in_specs=[pl.BlockSpec((BM, K), lambda i: (i, 0)),
                  pl.BlockSpec((2, K), lambda i: (0, 0)),
                  pl.BlockSpec((1, K), lambda i: (0, 0)),
                  pl.BlockSpec((1, K), lambda i: (0, 0)),
                  pl.BlockSpec((K, N), lambda i: (0, 0)),
                  pl.BlockSpec((1, N), lambda i: (0, 0))],
        out_specs=[pl.BlockSpec((BM, N), lambda i: (i, 0)),
                   pl.BlockSpec((2, N), lambda i: (0, 0))],
        out_shape=[jax.ShapeDtypeStruct((Bn, N), jnp.float32),
                   jax.ShapeDtypeStruct((2, N), jnp.float32)],
    )(z, st, g, be, w, b)


def _final_body(zu_ref, stu_ref, gu_ref, beu_ref,
                zi_ref, sti_ref, gi_ref, bei_ref, sw_ref, y_ref):
    u = jnp.maximum(_bn(zu_ref[...], stu_ref[...], gu_ref[...], beu_ref[...]),
                    0.0)
    v = jnp.maximum(_bn(zi_ref[...], sti_ref[...], gi_ref[...], bei_ref[...]),
                    0.0)
    dot = jnp.sum(u * v, axis=1, keepdims=True)
    un = jnp.sqrt(jnp.sum(u * u, axis=1, keepdims=True))
    vn = jnp.sqrt(jnp.sum(v * v, axis=1, keepdims=True))
    y_ref[...] = dot / jnp.maximum(un * vn, 1e-8) - jnp.log(sw_ref[...])


def _final(zu, stu, gu, beu, zi, sti, gi, bei, sw):
    Ku = zu.shape[1]
    Ki = zi.shape[1]
    return pl.pallas_call(
        _final_body,
        grid=(B // BM,),
        in_specs=[pl.BlockSpec((BM, Ku), lambda i: (i, 0)),
                  pl.BlockSpec((2, Ku), lambda i: (0, 0)),
                  pl.BlockSpec((1, Ku), lambda i: (0, 0)),
                  pl.BlockSpec((1, Ku), lambda i: (0, 0)),
                  pl.BlockSpec((BM, Ki), lambda i: (i, 0)),
                  pl.BlockSpec((2, Ki), lambda i: (0, 0)),
                  pl.BlockSpec((1, Ki), lambda i: (0, 0)),
                  pl.BlockSpec((1, Ki), lambda i: (0, 0)),
                  pl.BlockSpec((BM, 1), lambda i: (i, 0))],
        out_specs=pl.BlockSpec((BM, 1), lambda i: (i, 0)),
        out_shape=jax.ShapeDtypeStruct((B, 1), jnp.float32),
    )(zu, stu, gu, beu, zi, sti, gi, bei, sw)


def _scores_body(y_ref, out_ref):
    y = y_ref[...]
    rows = [y]
    for j in range(1, 4):
        rows.append(jnp.concatenate([y[:, j:], y[:, :j]], axis=1))
    out_ref[...] = jnp.concatenate(rows, axis=0)


def _scores(y_row):
    return pl.pallas_call(
        _scores_body,
        out_shape=jax.ShapeDtypeStruct((4, B), jnp.float32),
    )(y_row)


def kernel(user_ids, item_ids, sample_weight, user_tables, item_tables,
           uW1, ub1, ug1, ube1, uW2, ub2, ug2, ube2,
           iW1, ib1, ig1, ibe1, iW2, ib2, ig2, ibe2):
    utab = user_tables.reshape(NU * V, D)
    itab = item_tables.reshape(NI * V, D)
    uidx = (user_ids.astype(jnp.int32)
            + (jnp.arange(NU, dtype=jnp.int32) * V)[None, :]
            ).reshape(U_ROWS // CH, CH)
    iidx = (item_ids.astype(jnp.int32)
            + (jnp.arange(NI, dtype=jnp.int32) * V)[None, :]
            ).reshape(I_ROWS // CH, CH)

    ue, ie = _gather_embeddings(utab, uidx, itab, iidx)
    xu = ue.reshape(B, NU * D)
    xi = ie.reshape(B, NI * D)

    z1u, s1u = _mm_stats(xu, uW1, ub1.reshape(1, -1))
    z2u, s2u = _bn_mm_stats(z1u, s1u, ug1.reshape(1, -1), ube1.reshape(1, -1),
                            uW2, ub2.reshape(1, -1))
    z1i, s1i = _mm_stats(xi, iW1, ib1.reshape(1, -1))
    z2i, s2i = _bn_mm_stats(z1i, s1i, ig1.reshape(1, -1), ibe1.reshape(1, -1),
                            iW2, ib2.reshape(1, -1))

    y = _final(z2u, s2u, ug2.reshape(1, -1), ube2.reshape(1, -1),
               z2i, s2i, ig2.reshape(1, -1), ibe2.reshape(1, -1),
               sample_weight.reshape(B, 1))

    sc4 = _scores(y.reshape(1, B))
    return sc4.T


# speedup vs baseline: 8.3106x; 1.0078x over previous
import functools

import jax
import jax.numpy as jnp
from jax import lax
from jax.experimental import pallas as pl
from jax.experimental.pallas import tpu as pltpu
from jax.experimental.pallas import tpu_sc as plsc

B = 16384
V = 100000
D = 32
NU = 8
NI = 4

NC = 2
NS = 16
NW = NC * NS

CH = 128
GRP = 8
NCH = B // CH
U_WPF = NW // NU
I_WPF = NW // NI
U_CPW = NCH // U_WPF
I_CPW = NCH // I_WPF

BM = 2048


def _sc_gather_body(utab, uidxT, itab, iidxT, xu, xi, idx_v, rows_v, sem):
    wid = lax.axis_index("s") * NC + lax.axis_index("c")

    def tower(tab, idxT, out_hbm, wpf, cpw):
        f = wid // wpf
        cb0 = (wid % wpf) * cpw

        def group(g, _):
            cb = cb0 + g * GRP
            pltpu.sync_copy(idxT.at[f, pl.ds(cb, GRP)], idx_v)
            cps = []
            for s in range(GRP):
                cps.append(pltpu.async_copy(tab.at[f].at[idx_v.at[s]],
                                            rows_v.at[s], sem))
            for cp in cps:
                cp.wait()
            for s in range(GRP):
                pltpu.sync_copy(
                    rows_v.at[s],
                    out_hbm.at[pl.ds((cb + s) * CH, CH), pl.ds(f * D, D)])
            return _

        lax.fori_loop(0, cpw // GRP, group, 0)

    tower(utab, uidxT, xu, U_WPF, U_CPW)
    tower(itab, iidxT, xi, I_WPF, I_CPW)


def _gather_embeddings(utab, uidxT, itab, iidxT):
    mesh = plsc.VectorSubcoreMesh(core_axis_name="c", subcore_axis_name="s")
    f = functools.partial(
        pl.kernel,
        out_type=[jax.ShapeDtypeStruct((B, NU * D), jnp.float32),
                  jax.ShapeDtypeStruct((B, NI * D), jnp.float32)],
        mesh=mesh,
        scratch_types=[pltpu.VMEM((GRP, CH), jnp.int32),
                       pltpu.VMEM((GRP, CH, D), jnp.float32),
                       pltpu.SemaphoreType.DMA],
        compiler_params=pltpu.CompilerParams(use_tc_tiling_on_sc=False),
    )(_sc_gather_body)
    return f(utab, uidxT, itab, iidxT)


def _mm_stats_body(x_ref, w_ref, b_ref, z_ref, st_ref):
    z = jnp.dot(x_ref[...], w_ref[...],
                preferred_element_type=jnp.float32) + b_ref[...]
    z_ref[...] = z

    @pl.when(pl.program_id(0) == 0)
    def _():
        st_ref[...] = jnp.zeros_like(st_ref)

    st_ref[...] += jnp.concatenate(
        [jnp.sum(z, axis=0, keepdims=True),
         jnp.sum(z * z, axis=0, keepdims=True)], axis=0)


def _mm_stats(x, w, b):
    Bn, K = x.shape
    N = w.shape[1]
    return pl.pallas_call(
        _mm_stats_body,
        grid=(Bn // BM,),
        in_specs=[pl.BlockSpec((BM, K), lambda i: (i, 0)),
                  pl.BlockSpec((K, N), lambda i: (0, 0)),
                  pl.BlockSpec((1, N), lambda i: (0, 0))],
        out_specs=[pl.BlockSpec((BM, N), lambda i: (i, 0)),
                   pl.BlockSpec((2, N), lambda i: (0, 0))],
        out_shape=[jax.ShapeDtypeStruct((Bn, N), jnp.float32),
                   jax.ShapeDtypeStruct((2, N), jnp.float32)],
    )(x, w, b)


def _bn(z, st, g, be):
    mu = st[0:1, :] * (1.0 / B)
    var = st[1:2, :] * (1.0 / B) - mu * mu
    return (z - mu) / jnp.sqrt(var + 1e-5) * g + be


def _bn_mm_stats_body(z_ref, st_ref, g_ref, be_ref, w_ref, b_ref,
                      z2_ref, st2_ref):
    h = jnp.maximum(_bn(z_ref[...], st_ref[...], g_ref[...], be_ref[...]), 0.0)
    z2 = jnp.dot(h, w_ref[...],
                 preferred_element_type=jnp.float32) + b_ref[...]
    z2_ref[...] = z2

    @pl.when(pl.program_id(0) == 0)
    def _():
        st2_ref[...] = jnp.zeros_like(st2_ref)

    st2_ref[...] += jnp.concatenate(
        [jnp.sum(z2, axis=0, keepdims=True),
         jnp.sum(z2 * z2, axis=0, keepdims=True)], axis=0)


def _bn_mm_stats(z, st, g, be, w, b):
    Bn, K = z.shape
    N = w.shape[1]
    return pl.pallas_call(
        _bn_mm_stats_body,
        grid=(Bn // BM,),
        in_specs=[pl.BlockSpec((BM, K), lambda i: (i, 0)),
                  pl.BlockSpec((2, K), lambda i: (0, 0)),
                  pl.BlockSpec((1, K), lambda i: (0, 0)),
                  pl.BlockSpec((1, K), lambda i: (0, 0)),
                  pl.BlockSpec((K, N), lambda i: (0, 0)),
                  pl.BlockSpec((1, N), lambda i: (0, 0))],
        out_specs=[pl.BlockSpec((BM, N), lambda i: (i, 0)),
                   pl.BlockSpec((2, N), lambda i: (0, 0))],
        out_shape=[jax.ShapeDtypeStruct((Bn, N), jnp.float32),
                   jax.ShapeDtypeStruct((2, N), jnp.float32)],
    )(z, st, g, be, w, b)


def _final_body(zu_ref, stu_ref, gu_ref, beu_ref,
                zi_ref, sti_ref, gi_ref, bei_ref, sw_ref, y_ref):
    u = jnp.maximum(_bn(zu_ref[...], stu_ref[...], gu_ref[...], beu_ref[...]),
                    0.0)
    v = jnp.maximum(_bn(zi_ref[...], sti_ref[...], gi_ref[...], bei_ref[...]),
                    0.0)
    dot = jnp.sum(u * v, axis=1, keepdims=True)
    un = jnp.sqrt(jnp.sum(u * u, axis=1, keepdims=True))
    vn = jnp.sqrt(jnp.sum(v * v, axis=1, keepdims=True))
    y_ref[...] = dot / jnp.maximum(un * vn, 1e-8) - jnp.log(sw_ref[...])


def _final(zu, stu, gu, beu, zi, sti, gi, bei, sw):
    Ku = zu.shape[1]
    Ki = zi.shape[1]
    return pl.pallas_call(
        _final_body,
        grid=(B // BM,),
        in_specs=[pl.BlockSpec((BM, Ku), lambda i: (i, 0)),
                  pl.BlockSpec((2, Ku), lambda i: (0, 0)),
                  pl.BlockSpec((1, Ku), lambda i: (0, 0)),
                  pl.BlockSpec((1, Ku), lambda i: (0, 0)),
                  pl.BlockSpec((BM, Ki), lambda i: (i, 0)),
                  pl.BlockSpec((2, Ki), lambda i: (0, 0)),
                  pl.BlockSpec((1, Ki), lambda i: (0, 0)),
                  pl.BlockSpec((1, Ki), lambda i: (0, 0)),
                  pl.BlockSpec((BM, 1), lambda i: (i, 0))],
        out_specs=pl.BlockSpec((BM, 1), lambda i: (i, 0)),
        out_shape=jax.ShapeDtypeStruct((B, 1), jnp.float32),
    )(zu, stu, gu, beu, zi, sti, gi, bei, sw)


def _scores_body(y_ref, out_ref):
    y = y_ref[...]
    rows = [y]
    for j in range(1, 4):
        rows.append(jnp.concatenate([y[:, j:], y[:, :j]], axis=1))
    out_ref[...] = jnp.concatenate(rows, axis=0)


def _scores(y_row):
    return pl.pallas_call(
        _scores_body,
        out_shape=jax.ShapeDtypeStruct((4, B), jnp.float32),
    )(y_row)


def kernel(user_ids, item_ids, sample_weight, user_tables, item_tables,
           uW1, ub1, ug1, ube1, uW2, ub2, ug2, ube2,
           iW1, ib1, ig1, ibe1, iW2, ib2, ig2, ibe2):
    uidxT = user_ids.astype(jnp.int32).T.reshape(NU, NCH, CH)
    iidxT = item_ids.astype(jnp.int32).T.reshape(NI, NCH, CH)

    xu, xi = _gather_embeddings(user_tables, uidxT, item_tables, iidxT)

    z1u, s1u = _mm_stats(xu, uW1, ub1.reshape(1, -1))
    z2u, s2u = _bn_mm_stats(z1u, s1u, ug1.reshape(1, -1), ube1.reshape(1, -1),
                            uW2, ub2.reshape(1, -1))
    z1i, s1i = _mm_stats(xi, iW1, ib1.reshape(1, -1))
    z2i, s2i = _bn_mm_stats(z1i, s1i, ig1.reshape(1, -1), ibe1.reshape(1, -1),
                            iW2, ib2.reshape(1, -1))

    y = _final(z2u, s2u, ug2.reshape(1, -1), ube2.reshape(1, -1),
               z2i, s2i, ig2.reshape(1, -1), ibe2.reshape(1, -1),
               sample_weight.reshape(B, 1))

    sc4 = _scores(y.reshape(1, B))
    return sc4.T
